# Initial kernel scaffold; baseline (speedup 1.0000x reference)
import functools

import jax
import jax.numpy as jnp
from jax import lax
from jax.experimental import pallas as pl
from jax.experimental.pallas import tpu as pltpu
from jax.experimental.pallas import tpu_sc as plsc

N = 10000
E = 320000
C = 128
R = 16

NUM_CORES = 2
NUM_SUBCORES = 16
NW = NUM_CORES * NUM_SUBCORES
CH = 128
NUM_CHUNKS = E // CH
CHUNKS_PER_W = -(-NUM_CHUNKS // NW)
ROWS_PER_TILE = N // NUM_SUBCORES
S_CHUNK = 624


def _sc_edge_body(head_h, tail_h, type_h, x_h, uv_h, par_h,
                  agg_o, s_o,
                  zrow_v, zmat_v, h_v, t_v, ty_v, uvh_v, uvt_v, xr_v, ex_v,
                  par_v, sem, agg_sh, s_sh):
    cid_ax = lax.axis_index("c")
    sid = lax.axis_index("s")
    wid = sid * NUM_CORES + cid_ax

    def _zrow(i, carry):
        zrow_v[pl.ds(i * 16, 16)] = jnp.zeros((16,), jnp.float32)
        return carry

    lax.fori_loop(0, 40, _zrow, 0)

    def _zmat(i, carry):
        for k in range(C // 16):
            zmat_v[i, pl.ds(k * 16, 16)] = jnp.zeros((16,), jnp.float32)
        return carry

    lax.fori_loop(0, 125, _zmat, 0)

    for k in range(5):
        pltpu.sync_copy(zmat_v, agg_sh.at[pl.ds(sid * ROWS_PER_TILE + k * 125, 125)])
    pltpu.sync_copy(zrow_v.at[pl.ds(0, S_CHUNK)], s_sh.at[pl.ds(sid * S_CHUNK, S_CHUNK)])

    @pl.when(sid == NUM_SUBCORES - 1)
    def _():
        pltpu.sync_copy(zrow_v.at[pl.ds(0, 16)],
                        s_sh.at[pl.ds(NUM_SUBCORES * S_CHUNK, N - NUM_SUBCORES * S_CHUNK)])

    pltpu.sync_copy(par_h, par_v)
    plsc.subcore_barrier()

    bvec = par_v[1, :]

    def _chunk(i, carry):
        cid = wid + i * NW

        @pl.when(cid < NUM_CHUNKS)
        def _():
            base = cid * CH
            pltpu.sync_copy(head_h.at[pl.ds(base, CH)], h_v)
            pltpu.sync_copy(tail_h.at[pl.ds(base, CH)], t_v)
            pltpu.sync_copy(type_h.at[pl.ds(base, CH)], ty_v)
            cp1 = pltpu.async_copy(uv_h.at[h_v], uvh_v, sem)
            cp2 = pltpu.async_copy(uv_h.at[t_v], uvt_v, sem)
            cp3 = pltpu.async_copy(x_h.at[t_v], xr_v, sem)
            cp1.wait()
            cp2.wait()
            cp3.wait()

            zero16 = jnp.zeros((16,), jnp.int32)
            for g in range(CH // 16):
                rows = lax.iota(jnp.int32, 16) + (g * 16)
                ty16 = ty_v[pl.ds(g * 16, 16)]
                u16 = plsc.load_gather(uvh_v, [rows, ty16])
                v16 = plsc.load_gather(uvt_v, [rows, ty16 + R])
                b16 = plsc.load_gather(par_v, [zero16, ty16])
                ein = u16 + v16 + b16
                e = jnp.where(ein >= 0.0, ein, ein * 0.2)
                ex_v[pl.ds(g * 16, 16)] = jnp.exp(e - bvec)

            def _scale(j, carry2):
                a = plsc.load_gather(ex_v, [jnp.full((16,), j, jnp.int32)])
                for k in range(C // 16):
                    xr_v[j, pl.ds(k * 16, 16)] = xr_v[j, pl.ds(k * 16, 16)] * a
                return carry2

            lax.fori_loop(0, CH, _scale, 0)

            pltpu.sync_copy(xr_v, agg_sh.at[h_v], add=True)
            pltpu.sync_copy(ex_v, s_sh.at[h_v], add=True)

        return carry

    lax.fori_loop(0, CHUNKS_PER_W, _chunk, 0)
    plsc.subcore_barrier()

    pltpu.sync_copy(agg_sh.at[pl.ds(sid * ROWS_PER_TILE, ROWS_PER_TILE)],
                    agg_o.at[cid_ax, pl.ds(sid * ROWS_PER_TILE, ROWS_PER_TILE)])
    pltpu.sync_copy(s_sh.at[pl.ds(sid * S_CHUNK, S_CHUNK)],
                    s_o.at[cid_ax, pl.ds(sid * S_CHUNK, S_CHUNK)])

    @pl.when(sid == NUM_SUBCORES - 1)
    def _():
        pltpu.sync_copy(s_sh.at[pl.ds(NUM_SUBCORES * S_CHUNK, N - NUM_SUBCORES * S_CHUNK)],
                        s_o.at[cid_ax, pl.ds(NUM_SUBCORES * S_CHUNK, N - NUM_SUBCORES * S_CHUNK)])


_sc_edge = functools.partial(
    pl.kernel,
    mesh=plsc.VectorSubcoreMesh(core_axis_name="c", subcore_axis_name="s"),
    out_type=[
        jax.ShapeDtypeStruct((NUM_CORES, N, C), jnp.float32),
        jax.ShapeDtypeStruct((NUM_CORES, N), jnp.float32),
    ],
    scratch_types=[
        pltpu.VMEM((640,), jnp.float32),
        pltpu.VMEM((125, C), jnp.float32),
        pltpu.VMEM((CH,), jnp.int32),
        pltpu.VMEM((CH,), jnp.int32),
        pltpu.VMEM((CH,), jnp.int32),
        pltpu.VMEM((CH, 2 * R), jnp.float32),
        pltpu.VMEM((CH, 2 * R), jnp.float32),
        pltpu.VMEM((CH, C), jnp.float32),
        pltpu.VMEM((CH,), jnp.float32),
        pltpu.VMEM((2, R), jnp.float32),
        pltpu.SemaphoreType.DMA,
        pltpu.VMEM_SHARED((N, C), jnp.float32),
        pltpu.VMEM_SHARED((N,), jnp.float32),
    ],
)(_sc_edge_body)


def _prologue_body(x_ref, rel_ref, fw_ref, fb_ref,
                   uv_ref, par_ref, w_ref_o, bc_ref_o):
    rel = rel_ref[...]
    fw = fw_ref[...]
    wh = jnp.dot(rel, fw[:, :C], preferred_element_type=jnp.float32)
    wt = jnp.dot(rel, fw[:, C:], preferred_element_type=jnp.float32)
    w = jnp.concatenate([wh, wt], axis=0).T
    bc = jnp.dot(rel, fb_ref[...].reshape(C, 1),
                 preferred_element_type=jnp.float32).T
    uv = jnp.dot(x_ref[...], w, preferred_element_type=jnp.float32)
    uv_ref[...] = uv
    w_ref_o[...] = w
    bc_ref_o[...] = bc
    b = jnp.max(uv[:, :R]) + jnp.max(uv[:, R:]) + jnp.max(bc)
    par_ref[0:1, :] = bc
    par_ref[1:2, :] = jnp.full((1, R), 0.0, jnp.float32) + b


_prologue = pl.pallas_call(
    _prologue_body,
    out_shape=[
        jax.ShapeDtypeStruct((N, 2 * R), jnp.float32),
        jax.ShapeDtypeStruct((2, R), jnp.float32),
        jax.ShapeDtypeStruct((C, 2 * R), jnp.float32),
        jax.ShapeDtypeStruct((1, R), jnp.float32),
    ],
)


def _update_body(with_uv, agg_ref, s_ref, x_ref, w_ref, bc_ref, xo_ref, *rest):
    a = agg_ref[...]
    s = s_ref[...]
    ssum = s[0] + s[1]
    agg = a[0] + a[1]
    denom = jnp.where(ssum > 0.0, ssum, 1.0)
    row = jnp.where((ssum > 0.0)[:, None], agg / denom[:, None], 0.0) + x_ref[...]
    nrm = jnp.sqrt(jnp.sum(row * row, axis=1, keepdims=True))
    xo = row / jnp.maximum(nrm, 1e-12)
    xo_ref[...] = xo
    if with_uv:
        uv_ref, par_ref = rest
        uv = jnp.dot(xo, w_ref[...], preferred_element_type=jnp.float32)
        uv_ref[...] = uv
        bc = bc_ref[...]
        b = jnp.max(uv[:, :R]) + jnp.max(uv[:, R:]) + jnp.max(bc)
        par_ref[0:1, :] = bc
        par_ref[1:2, :] = jnp.full((1, R), 0.0, jnp.float32) + b


_update_mid = pl.pallas_call(
    functools.partial(_update_body, True),
    out_shape=[
        jax.ShapeDtypeStruct((N, C), jnp.float32),
        jax.ShapeDtypeStruct((N, 2 * R), jnp.float32),
        jax.ShapeDtypeStruct((2, R), jnp.float32),
    ],
)

_update_last = pl.pallas_call(
    functools.partial(_update_body, False),
    out_shape=[jax.ShapeDtypeStruct((N, C), jnp.float32)],
)


def kernel(entity_emb, relation_emb, fc_w, fc_b, edge_index, edge_type):
    head = edge_index[0]
    tail = edge_index[1]

    uv, par, w, bc = _prologue(entity_emb, relation_emb, fc_w, fc_b)

    agg2, s2 = _sc_edge(head, tail, edge_type, entity_emb, uv, par)
    x1, uv, par = _update_mid(agg2, s2, entity_emb, w, bc)

    agg2, s2 = _sc_edge(head, tail, edge_type, x1, uv, par)
    (x2,) = _update_last(agg2, s2, x1, w, bc)
    return x2


# speedup vs baseline: 15.3796x; 15.3796x over previous
import functools

import jax
import jax.numpy as jnp
from jax import lax
from jax.experimental import pallas as pl
from jax.experimental.pallas import tpu as pltpu
from jax.experimental.pallas import tpu_sc as plsc

N = 10000
E = 320000
C = 128
R = 16

NUM_CORES = 2
NUM_SUBCORES = 16
NW = NUM_CORES * NUM_SUBCORES
CH = 128
NUM_CHUNKS = E // CH
CHUNKS_PER_W = -(-NUM_CHUNKS // NW)
S_CHUNK = 624
S_TAIL = N - NUM_SUBCORES * S_CHUNK


def _sc_edge_body(head_h, tail_h, type_h, x_h, uvb_h, bvec_h,
                  agg_o, s_o,
                  zrow_v, zmat_v, h_v, t_v, ty_v, iu_v, iw_v, u_v, w_v,
                  xr_v, ex_v, bvec_v, sem, agg_sh, s_sh):
    cid_ax = lax.axis_index("c")
    sid = lax.axis_index("s")
    wid = sid * NUM_CORES + cid_ax

    def _zrow(i, carry):
        zrow_v[pl.ds(i * 16, 16)] = jnp.zeros((16,), jnp.float32)
        return carry

    lax.fori_loop(0, 40, _zrow, 0)

    def _zmat(i, carry):
        for k in range(C // 16):
            zmat_v[i, pl.ds(k * 16, 16)] = jnp.zeros((16,), jnp.float32)
        return carry

    lax.fori_loop(0, 104, _zmat, 0)

    for k in range(6):
        pltpu.sync_copy(zmat_v, agg_sh.at[pl.ds(sid * S_CHUNK + k * 104, 104)])
    pltpu.sync_copy(zrow_v.at[pl.ds(0, S_CHUNK)], s_sh.at[pl.ds(sid * S_CHUNK, S_CHUNK)])

    @pl.when(sid == NUM_SUBCORES - 1)
    def _():
        pltpu.sync_copy(zmat_v.at[pl.ds(0, S_TAIL)],
                        agg_sh.at[pl.ds(NUM_SUBCORES * S_CHUNK, S_TAIL)])
        pltpu.sync_copy(zrow_v.at[pl.ds(0, S_TAIL)],
                        s_sh.at[pl.ds(NUM_SUBCORES * S_CHUNK, S_TAIL)])

    pltpu.sync_copy(bvec_h, bvec_v)
    plsc.subcore_barrier()

    bvec = bvec_v[...]

    def _chunk(i, carry):
        cid = wid + i * NW

        @pl.when(cid < NUM_CHUNKS)
        def _():
            base = cid * CH
            pltpu.sync_copy(head_h.at[pl.ds(base, CH)], h_v)
            pltpu.sync_copy(tail_h.at[pl.ds(base, CH)], t_v)
            pltpu.sync_copy(type_h.at[pl.ds(base, CH)], ty_v)

            for g in range(CH // 16):
                ds = pl.ds(g * 16, 16)
                ty16 = ty_v[ds]
                iu_v[ds] = h_v[ds] * 32 + ty16
                iw_v[ds] = t_v[ds] * 32 + (ty16 + R)

            cp1 = pltpu.async_copy(uvb_h.at[iu_v], u_v, sem)
            cp2 = pltpu.async_copy(uvb_h.at[iw_v], w_v, sem)
            cp3 = pltpu.async_copy(x_h.at[t_v], xr_v, sem)
            cp1.wait()
            cp2.wait()
            cp3.wait()

            for g in range(CH // 16):
                ds = pl.ds(g * 16, 16)
                ein = u_v[ds] + w_v[ds]
                e = jnp.where(ein >= 0.0, ein, ein * 0.2)
                ex_v[ds] = jnp.exp(e - bvec)

            def _scale(g, carry2):
                ex16 = ex_v[pl.ds(g * 16, 16)]
                for j in range(16):
                    av = jnp.broadcast_to(ex16[j], (16,))
                    row = g * 16 + j
                    for k in range(C // 16):
                        xr_v[row, pl.ds(k * 16, 16)] = xr_v[row, pl.ds(k * 16, 16)] * av
                return carry2

            lax.fori_loop(0, CH // 16, _scale, 0)

            pltpu.sync_copy(xr_v, agg_sh.at[h_v], add=True)
            pltpu.sync_copy(ex_v, s_sh.at[h_v], add=True)

        return carry

    lax.fori_loop(0, CHUNKS_PER_W, _chunk, 0)
    plsc.subcore_barrier()

    pltpu.sync_copy(agg_sh.at[pl.ds(sid * S_CHUNK, S_CHUNK)],
                    agg_o.at[cid_ax, pl.ds(sid * S_CHUNK, S_CHUNK)])
    pltpu.sync_copy(s_sh.at[pl.ds(sid * S_CHUNK, S_CHUNK)], zrow_v.at[pl.ds(0, S_CHUNK)])
    pltpu.sync_copy(zrow_v.at[pl.ds(0, S_CHUNK)],
                    s_o.at[pl.ds(cid_ax * N + sid * S_CHUNK, S_CHUNK)])

    @pl.when(sid == NUM_SUBCORES - 1)
    def _():
        pltpu.sync_copy(agg_sh.at[pl.ds(NUM_SUBCORES * S_CHUNK, S_TAIL)],
                        agg_o.at[cid_ax, pl.ds(NUM_SUBCORES * S_CHUNK, S_TAIL)])
        pltpu.sync_copy(s_sh.at[pl.ds(NUM_SUBCORES * S_CHUNK, S_TAIL)],
                        zrow_v.at[pl.ds(0, S_TAIL)])
        pltpu.sync_copy(zrow_v.at[pl.ds(0, S_TAIL)],
                        s_o.at[pl.ds(cid_ax * N + NUM_SUBCORES * S_CHUNK, S_TAIL)])


_sc_edge = functools.partial(
    pl.kernel,
    mesh=plsc.VectorSubcoreMesh(core_axis_name="c", subcore_axis_name="s"),
    out_type=[
        jax.ShapeDtypeStruct((NUM_CORES, N, C), jnp.float32),
        jax.ShapeDtypeStruct((NUM_CORES * N,), jnp.float32),
    ],
    scratch_types=[
        pltpu.VMEM((640,), jnp.float32),
        pltpu.VMEM((104, C), jnp.float32),
        pltpu.VMEM((CH,), jnp.int32),
        pltpu.VMEM((CH,), jnp.int32),
        pltpu.VMEM((CH,), jnp.int32),
        pltpu.VMEM((CH,), jnp.int32),
        pltpu.VMEM((CH,), jnp.int32),
        pltpu.VMEM((CH,), jnp.float32),
        pltpu.VMEM((CH,), jnp.float32),
        pltpu.VMEM((CH, C), jnp.float32),
        pltpu.VMEM((CH,), jnp.float32),
        pltpu.VMEM((R,), jnp.float32),
        pltpu.SemaphoreType.DMA,
        pltpu.VMEM_SHARED((N, C), jnp.float32),
        pltpu.VMEM_SHARED((N,), jnp.float32),
    ],
)(_sc_edge_body)


def _prologue_body(x_ref, rel_ref, fw_ref, fb_ref,
                   uvb_ref, bv_ref, w_ref_o, bcp_ref_o):
    rel = rel_ref[...]
    fw = fw_ref[...]
    wh = jnp.dot(rel, fw[:, :C], preferred_element_type=jnp.float32)
    wt = jnp.dot(rel, fw[:, C:], preferred_element_type=jnp.float32)
    w = jnp.concatenate([wh, wt], axis=0).T
    bc = jnp.dot(rel, fb_ref[...].reshape(C, 1),
                 preferred_element_type=jnp.float32).T
    bcp = jnp.concatenate([bc, jnp.zeros((1, R), jnp.float32)], axis=1)
    uvb = jnp.dot(x_ref[...], w, preferred_element_type=jnp.float32) + bcp
    uvb_ref[...] = uvb
    w_ref_o[...] = w
    bcp_ref_o[...] = bcp
    b = jnp.max(uvb[:, :R]) + jnp.max(uvb[:, R:])
    bv_ref[...] = jnp.full((1, R), 0.0, jnp.float32) + b


_prologue = pl.pallas_call(
    _prologue_body,
    out_shape=[
        jax.ShapeDtypeStruct((N, 2 * R), jnp.float32),
        jax.ShapeDtypeStruct((1, R), jnp.float32),
        jax.ShapeDtypeStruct((C, 2 * R), jnp.float32),
        jax.ShapeDtypeStruct((1, 2 * R), jnp.float32),
    ],
)


def _update_body(with_uv, agg_ref, s_ref, x_ref, w_ref, bcp_ref, xo_ref, *rest):
    a = agg_ref[...]
    s = s_ref[...]
    ssum = s[0] + s[1]
    agg = a[0] + a[1]
    denom = jnp.where(ssum > 0.0, ssum, 1.0)
    row = jnp.where(ssum > 0.0, agg / denom, 0.0) + x_ref[...]
    nrm = jnp.sqrt(jnp.sum(row * row, axis=1, keepdims=True))
    xo = row / jnp.maximum(nrm, 1e-12)
    xo_ref[...] = xo
    if with_uv:
        uvb_ref, bv_ref = rest
        uvb = jnp.dot(xo, w_ref[...], preferred_element_type=jnp.float32) + bcp_ref[...]
        uvb_ref[...] = uvb
        b = jnp.max(uvb[:, :R]) + jnp.max(uvb[:, R:])
        bv_ref[...] = jnp.full((1, R), 0.0, jnp.float32) + b


_update_mid = pl.pallas_call(
    functools.partial(_update_body, True),
    out_shape=[
        jax.ShapeDtypeStruct((N, C), jnp.float32),
        jax.ShapeDtypeStruct((N, 2 * R), jnp.float32),
        jax.ShapeDtypeStruct((1, R), jnp.float32),
    ],
)

_update_last = pl.pallas_call(
    functools.partial(_update_body, False),
    out_shape=[jax.ShapeDtypeStruct((N, C), jnp.float32)],
)


def kernel(entity_emb, relation_emb, fc_w, fc_b, edge_index, edge_type):
    head = edge_index[0]
    tail = edge_index[1]

    uvb, bv, w, bcp = _prologue(entity_emb, relation_emb, fc_w, fc_b)

    agg2, s2 = _sc_edge(head, tail, edge_type, entity_emb,
                        uvb.reshape(N * 2 * R), bv.reshape(R))
    x1, uvb, bv = _update_mid(agg2, s2.reshape(NUM_CORES, N, 1), entity_emb, w, bcp)

    agg2, s2 = _sc_edge(head, tail, edge_type, x1,
                        uvb.reshape(N * 2 * R), bv.reshape(R))
    (x2,) = _update_last(agg2, s2.reshape(NUM_CORES, N, 1), x1, w, bcp)
    return x2


# speedup vs baseline: 22.2211x; 1.4448x over previous
import functools

import jax
import jax.numpy as jnp
from jax import lax
from jax.experimental import pallas as pl
from jax.experimental.pallas import tpu as pltpu
from jax.experimental.pallas import tpu_sc as plsc

N = 10000
E = 320000
C = 128
R = 16

NUM_CORES = 2
NUM_SUBCORES = 16
NW = NUM_CORES * NUM_SUBCORES
CH = 128
NUM_CHUNKS = E // CH
CHUNKS_PER_W = -(-NUM_CHUNKS // NW)
S_CHUNK = 624
S_TAIL = N - NUM_SUBCORES * S_CHUNK


def _sc_edge_body(head_h, tail_h, type_h, x_h, uvb_h, bvec_h,
                  agg_o, s_o,
                  zrow_v, zmat_v, h_v, t_v, ty_v, iu_v, iw_v, u_v, w_v,
                  xr_v, ex_v, bvec_v, sem_g0, sem_g1, sem_s0, sem_s1,
                  agg_sh, s_sh):
    cid_ax = lax.axis_index("c")
    sid = lax.axis_index("s")
    wid = sid * NUM_CORES + cid_ax

    def _zrow(i, carry):
        zrow_v[pl.ds(i * 16, 16)] = jnp.zeros((16,), jnp.float32)
        return carry

    lax.fori_loop(0, 40, _zrow, 0)

    def _zmat(i, carry):
        for k in range(C // 16):
            zmat_v[i, pl.ds(k * 16, 16)] = jnp.zeros((16,), jnp.float32)
        return carry

    lax.fori_loop(0, 104, _zmat, 0)

    for k in range(6):
        pltpu.sync_copy(zmat_v, agg_sh.at[pl.ds(sid * S_CHUNK + k * 104, 104)])
    pltpu.sync_copy(zrow_v.at[pl.ds(0, S_CHUNK)], s_sh.at[pl.ds(sid * S_CHUNK, S_CHUNK)])

    @pl.when(sid == NUM_SUBCORES - 1)
    def _():
        pltpu.sync_copy(zmat_v.at[pl.ds(0, S_TAIL)],
                        agg_sh.at[pl.ds(NUM_SUBCORES * S_CHUNK, S_TAIL)])
        pltpu.sync_copy(zrow_v.at[pl.ds(0, S_TAIL)],
                        s_sh.at[pl.ds(NUM_SUBCORES * S_CHUNK, S_TAIL)])

    pltpu.sync_copy(bvec_h, bvec_v)
    plsc.subcore_barrier()

    bvec = bvec_v[...]
    sems_g = (sem_g0, sem_g1)
    sems_s = (sem_s0, sem_s1)

    def _load_idx(cid, nb):
        base = cid * CH
        pltpu.sync_copy(head_h.at[pl.ds(base, CH)], h_v.at[nb])
        pltpu.sync_copy(tail_h.at[pl.ds(base, CH)], t_v.at[nb])
        pltpu.sync_copy(type_h.at[pl.ds(base, CH)], ty_v.at[nb])
        for g in range(CH // 16):
            ds = pl.ds(g * 16, 16)
            ty16 = ty_v[nb, ds]
            iu_v[nb, ds] = h_v[nb, ds] * 32 + ty16
            iw_v[nb, ds] = t_v[nb, ds] * 32 + (ty16 + R)

    def _fire_gathers(nb):
        pltpu.async_copy(uvb_h.at[iu_v.at[nb]], u_v.at[nb], sems_g[nb])
        pltpu.async_copy(uvb_h.at[iw_v.at[nb]], w_v.at[nb], sems_g[nb])
        pltpu.async_copy(x_h.at[t_v.at[nb]], xr_v.at[nb], sems_g[nb])

    _load_idx(wid, 0)
    _fire_gathers(0)

    def _pair(i2, carry):
        for b in (0, 1):
            k = i2 * 2 + b
            nb = 1 - b
            cid = wid + k * NW

            @pl.when((k >= 1) & (cid - NW < NUM_CHUNKS))
            def _():
                pltpu.make_async_copy(
                    xr_v.at[nb], agg_sh.at[h_v.at[nb]], sems_s[nb]).wait()
                pltpu.make_async_copy(
                    ex_v.at[nb], s_sh.at[h_v.at[nb]], sems_s[nb]).wait()

            @pl.when(cid + NW < NUM_CHUNKS)
            def _():
                _load_idx(cid + NW, nb)
                _fire_gathers(nb)

            @pl.when(cid < NUM_CHUNKS)
            def _():
                pltpu.make_async_copy(uvb_h.at[iu_v.at[b]], u_v.at[b], sems_g[b]).wait()
                pltpu.make_async_copy(uvb_h.at[iw_v.at[b]], w_v.at[b], sems_g[b]).wait()
                pltpu.make_async_copy(x_h.at[t_v.at[b]], xr_v.at[b], sems_g[b]).wait()

                for g in range(CH // 16):
                    ds = pl.ds(g * 16, 16)
                    ein = u_v[b, ds] + w_v[b, ds]
                    e = jnp.where(ein >= 0.0, ein, ein * 0.2)
                    ex_v[b, ds] = jnp.exp(e - bvec)

                def _scale(g, carry2):
                    ex16 = ex_v[b, pl.ds(g * 16, 16)]
                    for j in range(16):
                        av = jnp.broadcast_to(ex16[j], (16,))
                        row = g * 16 + j
                        for kk in range(C // 16):
                            xr_v[b, row, pl.ds(kk * 16, 16)] = (
                                xr_v[b, row, pl.ds(kk * 16, 16)] * av)
                    return carry2

                lax.fori_loop(0, CH // 16, _scale, 0)

                pltpu.async_copy(xr_v.at[b], agg_sh.at[h_v.at[b]], sems_s[b], add=True)
                pltpu.async_copy(ex_v.at[b], s_sh.at[h_v.at[b]], sems_s[b], add=True)

        return carry

    lax.fori_loop(0, (CHUNKS_PER_W + 2) // 2, _pair, 0)
    plsc.subcore_barrier()

    pltpu.sync_copy(agg_sh.at[pl.ds(sid * S_CHUNK, S_CHUNK)],
                    agg_o.at[cid_ax, pl.ds(sid * S_CHUNK, S_CHUNK)])
    pltpu.sync_copy(s_sh.at[pl.ds(sid * S_CHUNK, S_CHUNK)], zrow_v.at[pl.ds(0, S_CHUNK)])
    pltpu.sync_copy(zrow_v.at[pl.ds(0, S_CHUNK)],
                    s_o.at[pl.ds(cid_ax * N + sid * S_CHUNK, S_CHUNK)])

    @pl.when(sid == NUM_SUBCORES - 1)
    def _():
        pltpu.sync_copy(agg_sh.at[pl.ds(NUM_SUBCORES * S_CHUNK, S_TAIL)],
                        agg_o.at[cid_ax, pl.ds(NUM_SUBCORES * S_CHUNK, S_TAIL)])
        pltpu.sync_copy(s_sh.at[pl.ds(NUM_SUBCORES * S_CHUNK, S_TAIL)],
                        zrow_v.at[pl.ds(0, S_TAIL)])
        pltpu.sync_copy(zrow_v.at[pl.ds(0, S_TAIL)],
                        s_o.at[pl.ds(cid_ax * N + NUM_SUBCORES * S_CHUNK, S_TAIL)])


_sc_edge = functools.partial(
    pl.kernel,
    mesh=plsc.VectorSubcoreMesh(core_axis_name="c", subcore_axis_name="s"),
    out_type=[
        jax.ShapeDtypeStruct((NUM_CORES, N, C), jnp.float32),
        jax.ShapeDtypeStruct((NUM_CORES * N,), jnp.float32),
    ],
    scratch_types=[
        pltpu.VMEM((640,), jnp.float32),
        pltpu.VMEM((104, C), jnp.float32),
        pltpu.VMEM((2, CH), jnp.int32),
        pltpu.VMEM((2, CH), jnp.int32),
        pltpu.VMEM((2, CH), jnp.int32),
        pltpu.VMEM((2, CH), jnp.int32),
        pltpu.VMEM((2, CH), jnp.int32),
        pltpu.VMEM((2, CH), jnp.float32),
        pltpu.VMEM((2, CH), jnp.float32),
        pltpu.VMEM((2, CH, C), jnp.float32),
        pltpu.VMEM((2, CH), jnp.float32),
        pltpu.VMEM((R,), jnp.float32),
        pltpu.SemaphoreType.DMA,
        pltpu.SemaphoreType.DMA,
        pltpu.SemaphoreType.DMA,
        pltpu.SemaphoreType.DMA,
        pltpu.VMEM_SHARED((N, C), jnp.float32),
        pltpu.VMEM_SHARED((N,), jnp.float32),
    ],
)(_sc_edge_body)


def _prologue_body(x_ref, rel_ref, fw_ref, fb_ref,
                   uvb_ref, bv_ref, w_ref_o, bcp_ref_o):
    rel = rel_ref[...]
    fw = fw_ref[...]
    wh = jnp.dot(rel, fw[:, :C], preferred_element_type=jnp.float32)
    wt = jnp.dot(rel, fw[:, C:], preferred_element_type=jnp.float32)
    w = jnp.concatenate([wh, wt], axis=0).T
    bc = jnp.dot(rel, fb_ref[...].reshape(C, 1),
                 preferred_element_type=jnp.float32).T
    bcp = jnp.concatenate([bc, jnp.zeros((1, R), jnp.float32)], axis=1)
    uvb = jnp.dot(x_ref[...], w, preferred_element_type=jnp.float32) + bcp
    uvb_ref[...] = uvb
    w_ref_o[...] = w
    bcp_ref_o[...] = bcp
    b = jnp.max(uvb[:, :R]) + jnp.max(uvb[:, R:])
    bv_ref[...] = jnp.full((1, R), 0.0, jnp.float32) + b


_prologue = pl.pallas_call(
    _prologue_body,
    out_shape=[
        jax.ShapeDtypeStruct((N, 2 * R), jnp.float32),
        jax.ShapeDtypeStruct((1, R), jnp.float32),
        jax.ShapeDtypeStruct((C, 2 * R), jnp.float32),
        jax.ShapeDtypeStruct((1, 2 * R), jnp.float32),
    ],
)


def _update_body(with_uv, agg_ref, s_ref, x_ref, w_ref, bcp_ref, xo_ref, *rest):
    a = agg_ref[...]
    s = s_ref[...]
    ssum = s[0] + s[1]
    agg = a[0] + a[1]
    denom = jnp.where(ssum > 0.0, ssum, 1.0)
    row = jnp.where(ssum > 0.0, agg / denom, 0.0) + x_ref[...]
    nrm = jnp.sqrt(jnp.sum(row * row, axis=1, keepdims=True))
    xo = row / jnp.maximum(nrm, 1e-12)
    xo_ref[...] = xo
    if with_uv:
        uvb_ref, bv_ref = rest
        uvb = jnp.dot(xo, w_ref[...], preferred_element_type=jnp.float32) + bcp_ref[...]
        uvb_ref[...] = uvb
        b = jnp.max(uvb[:, :R]) + jnp.max(uvb[:, R:])
        bv_ref[...] = jnp.full((1, R), 0.0, jnp.float32) + b


_update_mid = pl.pallas_call(
    functools.partial(_update_body, True),
    out_shape=[
        jax.ShapeDtypeStruct((N, C), jnp.float32),
        jax.ShapeDtypeStruct((N, 2 * R), jnp.float32),
        jax.ShapeDtypeStruct((1, R), jnp.float32),
    ],
)

_update_last = pl.pallas_call(
    functools.partial(_update_body, False),
    out_shape=[jax.ShapeDtypeStruct((N, C), jnp.float32)],
)


def kernel(entity_emb, relation_emb, fc_w, fc_b, edge_index, edge_type):
    head = edge_index[0]
    tail = edge_index[1]

    uvb, bv, w, bcp = _prologue(entity_emb, relation_emb, fc_w, fc_b)

    agg2, s2 = _sc_edge(head, tail, edge_type, entity_emb,
                        uvb.reshape(N * 2 * R), bv.reshape(R))
    x1, uvb, bv = _update_mid(agg2, s2.reshape(NUM_CORES, N, 1), entity_emb, w, bcp)

    agg2, s2 = _sc_edge(head, tail, edge_type, x1,
                        uvb.reshape(N * 2 * R), bv.reshape(R))
    (x2,) = _update_last(agg2, s2.reshape(NUM_CORES, N, 1), x1, w, bcp)
    return x2


# speedup vs baseline: 29.7309x; 1.3380x over previous
import functools

import jax
import jax.numpy as jnp
from jax import lax
from jax.experimental import pallas as pl
from jax.experimental.pallas import tpu as pltpu
from jax.experimental.pallas import tpu_sc as plsc

N = 10000
E = 320000
C = 128
R = 16

NUM_CORES = 2
NUM_SUBCORES = 16
NW = NUM_CORES * NUM_SUBCORES
CH = 128
NUM_CHUNKS = E // CH
CHUNKS_PER_W = -(-NUM_CHUNKS // NW)
S_CHUNK = 624
S_TAIL = N - NUM_SUBCORES * S_CHUNK


def _sc_edge_body(epk_h, x_h, uvb_h, bvec_h,
                  agg_o, s_o,
                  zrow_v, zmat_v, e3_v, iu_v, iw_v, u_v, w_v,
                  xr_v, ex_v, bvec_v, sem_g0, sem_g1, sem_s0, sem_s1,
                  agg_sh, s_sh):
    cid_ax = lax.axis_index("c")
    sid = lax.axis_index("s")
    wid = sid * NUM_CORES + cid_ax

    def _zrow(i, carry):
        zrow_v[pl.ds(i * 16, 16)] = jnp.zeros((16,), jnp.float32)
        return carry

    lax.fori_loop(0, 40, _zrow, 0)

    def _zmat(i, carry):
        for k in range(C // 16):
            zmat_v[i, pl.ds(k * 16, 16)] = jnp.zeros((16,), jnp.float32)
        return carry

    lax.fori_loop(0, 104, _zmat, 0)

    for k in range(6):
        pltpu.async_copy(zmat_v, agg_sh.at[pl.ds(sid * S_CHUNK + k * 104, 104)], sem_g0)
    pltpu.async_copy(zrow_v.at[pl.ds(0, S_CHUNK)],
                     s_sh.at[pl.ds(sid * S_CHUNK, S_CHUNK)], sem_g0)

    @pl.when(sid == NUM_SUBCORES - 1)
    def _():
        pltpu.async_copy(zmat_v.at[pl.ds(0, S_TAIL)],
                         agg_sh.at[pl.ds(NUM_SUBCORES * S_CHUNK, S_TAIL)], sem_g0)
        pltpu.async_copy(zrow_v.at[pl.ds(0, S_TAIL)],
                         s_sh.at[pl.ds(NUM_SUBCORES * S_CHUNK, S_TAIL)], sem_g0)

    for k in range(6):
        pltpu.make_async_copy(
            zmat_v, agg_sh.at[pl.ds(sid * S_CHUNK + k * 104, 104)], sem_g0).wait()
    pltpu.make_async_copy(zrow_v.at[pl.ds(0, S_CHUNK)],
                          s_sh.at[pl.ds(sid * S_CHUNK, S_CHUNK)], sem_g0).wait()

    @pl.when(sid == NUM_SUBCORES - 1)
    def _():
        pltpu.make_async_copy(
            zmat_v.at[pl.ds(0, S_TAIL)],
            agg_sh.at[pl.ds(NUM_SUBCORES * S_CHUNK, S_TAIL)], sem_g0).wait()
        pltpu.make_async_copy(
            zrow_v.at[pl.ds(0, S_TAIL)],
            s_sh.at[pl.ds(NUM_SUBCORES * S_CHUNK, S_TAIL)], sem_g0).wait()

    pltpu.sync_copy(bvec_h, bvec_v)
    plsc.subcore_barrier()

    bvec = bvec_v[...]
    sems_g = (sem_g0, sem_g1)
    sems_s = (sem_s0, sem_s1)

    def _load_idx(cid, nb):
        pltpu.sync_copy(epk_h.at[cid, pl.ds(0, 3)], e3_v.at[nb])
        for g in range(CH // 16):
            ds = pl.ds(g * 16, 16)
            ty16 = e3_v[nb, 2, ds]
            iu_v[nb, ds] = e3_v[nb, 0, ds] * 32 + ty16
            iw_v[nb, ds] = e3_v[nb, 1, ds] * 32 + (ty16 + R)

    def _fire_gathers(nb):
        pltpu.async_copy(uvb_h.at[iu_v.at[nb]], u_v.at[nb], sems_g[nb])
        pltpu.async_copy(uvb_h.at[iw_v.at[nb]], w_v.at[nb], sems_g[nb])
        pltpu.async_copy(x_h.at[e3_v.at[nb, 1]], xr_v.at[nb], sems_g[nb])

    _load_idx(wid, 0)
    _fire_gathers(0)

    def _pair(i2, carry):
        for b in (0, 1):
            k = i2 * 2 + b
            nb = 1 - b
            cid = wid + k * NW

            @pl.when((k >= 1) & (cid - NW < NUM_CHUNKS))
            def _():
                pltpu.make_async_copy(
                    xr_v.at[nb], agg_sh.at[e3_v.at[nb, 0]], sems_s[nb]).wait()
                pltpu.make_async_copy(
                    ex_v.at[nb], s_sh.at[e3_v.at[nb, 0]], sems_s[nb]).wait()

            @pl.when(cid + NW < NUM_CHUNKS)
            def _():
                _load_idx(cid + NW, nb)
                _fire_gathers(nb)

            @pl.when(cid < NUM_CHUNKS)
            def _():
                pltpu.make_async_copy(uvb_h.at[iu_v.at[b]], u_v.at[b], sems_g[b]).wait()
                pltpu.make_async_copy(uvb_h.at[iw_v.at[b]], w_v.at[b], sems_g[b]).wait()
                pltpu.make_async_copy(x_h.at[e3_v.at[b, 1]], xr_v.at[b], sems_g[b]).wait()

                for g in range(CH // 16):
                    ds = pl.ds(g * 16, 16)
                    ein = u_v[b, ds] + w_v[b, ds]
                    e = jnp.where(ein >= 0.0, ein, ein * 0.2)
                    ex_v[b, ds] = jnp.exp(e - bvec)

                def _scale(g, carry2):
                    ex16 = ex_v[b, pl.ds(g * 16, 16)]
                    for j in range(16):
                        av = jnp.broadcast_to(ex16[j], (16,))
                        row = g * 16 + j
                        for kk in range(C // 16):
                            xr_v[b, row, pl.ds(kk * 16, 16)] = (
                                xr_v[b, row, pl.ds(kk * 16, 16)] * av)
                    return carry2

                lax.fori_loop(0, CH // 16, _scale, 0)

                pltpu.async_copy(xr_v.at[b], agg_sh.at[e3_v.at[b, 0]], sems_s[b], add=True)
                pltpu.async_copy(ex_v.at[b], s_sh.at[e3_v.at[b, 0]], sems_s[b], add=True)

        return carry

    lax.fori_loop(0, (CHUNKS_PER_W + 2) // 2, _pair, 0)
    plsc.subcore_barrier()

    pltpu.async_copy(agg_sh.at[pl.ds(sid * S_CHUNK, S_CHUNK)],
                     agg_o.at[cid_ax, pl.ds(sid * S_CHUNK, S_CHUNK)], sem_g0)
    pltpu.sync_copy(s_sh.at[pl.ds(sid * S_CHUNK, S_CHUNK)], zrow_v.at[pl.ds(0, S_CHUNK)])
    pltpu.sync_copy(zrow_v.at[pl.ds(0, S_CHUNK)],
                    s_o.at[pl.ds(cid_ax * N + sid * S_CHUNK, S_CHUNK)])
    pltpu.make_async_copy(agg_sh.at[pl.ds(sid * S_CHUNK, S_CHUNK)],
                          agg_o.at[cid_ax, pl.ds(sid * S_CHUNK, S_CHUNK)], sem_g0).wait()

    @pl.when(sid == NUM_SUBCORES - 1)
    def _():
        pltpu.sync_copy(agg_sh.at[pl.ds(NUM_SUBCORES * S_CHUNK, S_TAIL)],
                        agg_o.at[cid_ax, pl.ds(NUM_SUBCORES * S_CHUNK, S_TAIL)])
        pltpu.sync_copy(s_sh.at[pl.ds(NUM_SUBCORES * S_CHUNK, S_TAIL)],
                        zrow_v.at[pl.ds(0, S_TAIL)])
        pltpu.sync_copy(zrow_v.at[pl.ds(0, S_TAIL)],
                        s_o.at[pl.ds(cid_ax * N + NUM_SUBCORES * S_CHUNK, S_TAIL)])


_sc_edge = functools.partial(
    pl.kernel,
    mesh=plsc.VectorSubcoreMesh(core_axis_name="c", subcore_axis_name="s"),
    out_type=[
        jax.ShapeDtypeStruct((NUM_CORES, N, C), jnp.float32),
        jax.ShapeDtypeStruct((NUM_CORES * N,), jnp.float32),
    ],
    scratch_types=[
        pltpu.VMEM((640,), jnp.float32),
        pltpu.VMEM((104, C), jnp.float32),
        pltpu.VMEM((2, 3, CH), jnp.int32),
        pltpu.VMEM((2, CH), jnp.int32),
        pltpu.VMEM((2, CH), jnp.int32),
        pltpu.VMEM((2, CH), jnp.float32),
        pltpu.VMEM((2, CH), jnp.float32),
        pltpu.VMEM((2, CH, C), jnp.float32),
        pltpu.VMEM((2, CH), jnp.float32),
        pltpu.VMEM((R,), jnp.float32),
        pltpu.SemaphoreType.DMA,
        pltpu.SemaphoreType.DMA,
        pltpu.SemaphoreType.DMA,
        pltpu.SemaphoreType.DMA,
        pltpu.VMEM_SHARED((N, C), jnp.float32),
        pltpu.VMEM_SHARED((N,), jnp.float32),
    ],
)(_sc_edge_body)


def _prologue_body(x_ref, rel_ref, fw_ref, fb_ref,
                   uvb_ref, bv_ref, w_ref_o, bcp_ref_o):
    rel = rel_ref[...]
    fw = fw_ref[...]
    wh = jnp.dot(rel, fw[:, :C], preferred_element_type=jnp.float32)
    wt = jnp.dot(rel, fw[:, C:], preferred_element_type=jnp.float32)
    w = jnp.concatenate([wh, wt], axis=0).T
    bc = jnp.dot(rel, fb_ref[...].reshape(C, 1),
                 preferred_element_type=jnp.float32).T
    bcp = jnp.concatenate([bc, jnp.zeros((1, R), jnp.float32)], axis=1)
    uvb = jnp.dot(x_ref[...], w, preferred_element_type=jnp.float32) + bcp
    uvb_ref[...] = uvb
    w_ref_o[...] = w
    bcp_ref_o[...] = bcp
    b = jnp.max(uvb[:, :R]) + jnp.max(uvb[:, R:])
    bv_ref[...] = jnp.full((1, R), 0.0, jnp.float32) + b


_prologue = pl.pallas_call(
    _prologue_body,
    out_shape=[
        jax.ShapeDtypeStruct((N, 2 * R), jnp.float32),
        jax.ShapeDtypeStruct((1, R), jnp.float32),
        jax.ShapeDtypeStruct((C, 2 * R), jnp.float32),
        jax.ShapeDtypeStruct((1, 2 * R), jnp.float32),
    ],
)


def _update_body(with_uv, agg_ref, s_ref, x_ref, w_ref, bcp_ref, xo_ref, *rest):
    a = agg_ref[...]
    s = s_ref[...]
    ssum = s[0] + s[1]
    agg = a[0] + a[1]
    denom = jnp.where(ssum > 0.0, ssum, 1.0)
    row = jnp.where(ssum > 0.0, agg / denom, 0.0) + x_ref[...]
    nrm = jnp.sqrt(jnp.sum(row * row, axis=1, keepdims=True))
    xo = row / jnp.maximum(nrm, 1e-12)
    xo_ref[...] = xo
    if with_uv:
        uvb_ref, bv_ref = rest
        uvb = jnp.dot(xo, w_ref[...], preferred_element_type=jnp.float32) + bcp_ref[...]
        uvb_ref[...] = uvb
        b = jnp.max(uvb[:, :R]) + jnp.max(uvb[:, R:])
        bv_ref[...] = jnp.full((1, R), 0.0, jnp.float32) + b


_update_mid = pl.pallas_call(
    functools.partial(_update_body, True),
    out_shape=[
        jax.ShapeDtypeStruct((N, C), jnp.float32),
        jax.ShapeDtypeStruct((N, 2 * R), jnp.float32),
        jax.ShapeDtypeStruct((1, R), jnp.float32),
    ],
)

_update_last = pl.pallas_call(
    functools.partial(_update_body, False),
    out_shape=[jax.ShapeDtypeStruct((N, C), jnp.float32)],
)


def kernel(entity_emb, relation_emb, fc_w, fc_b, edge_index, edge_type):
    epk = jnp.concatenate(
        [edge_index.reshape(2, NUM_CHUNKS, 1, CH).transpose(1, 0, 2, 3)
         .reshape(NUM_CHUNKS, 2, CH),
         edge_type.reshape(NUM_CHUNKS, 1, CH),
         jnp.zeros((NUM_CHUNKS, 5, CH), jnp.int32)], axis=1)

    uvb, bv, w, bcp = _prologue(entity_emb, relation_emb, fc_w, fc_b)

    agg2, s2 = _sc_edge(epk, entity_emb, uvb.reshape(N * 2 * R), bv.reshape(R))
    x1, uvb, bv = _update_mid(agg2, s2.reshape(NUM_CORES, N, 1), entity_emb, w, bcp)

    agg2, s2 = _sc_edge(epk, x1, uvb.reshape(N * 2 * R), bv.reshape(R))
    (x2,) = _update_last(agg2, s2.reshape(NUM_CORES, N, 1), x1, w, bcp)
    return x2


# speedup vs baseline: 34.0123x; 1.1440x over previous
import functools

import jax
import jax.numpy as jnp
from jax import lax
from jax.experimental import pallas as pl
from jax.experimental.pallas import tpu as pltpu
from jax.experimental.pallas import tpu_sc as plsc

N = 10000
E = 320000
C = 128
R = 16

NUM_CORES = 2
NUM_SUBCORES = 16
NW = NUM_CORES * NUM_SUBCORES
CH = 128
NUM_CHUNKS = E // CH
CHUNKS_PER_W = -(-NUM_CHUNKS // NW)
S_CHUNK = 624
S_TAIL = N - NUM_SUBCORES * S_CHUNK


def _sc_edge_body(epk_h, x_h, uvb_h, bvec_h,
                  agg_o, s_o,
                  zrow_v, zmat_v, e3_v, hsc_v, iu_v, iw_v, u_v, w_v,
                  xr_v, ex_v, bvec_v, sem_g0, sem_g1, sem_s0, sem_s1,
                  sem_i0, sem_i1, agg_sh, s_sh):
    cid_ax = lax.axis_index("c")
    sid = lax.axis_index("s")
    wid = sid * NUM_CORES + cid_ax

    def _zrow(i, carry):
        zrow_v[pl.ds(i * 16, 16)] = jnp.zeros((16,), jnp.float32)
        return carry

    lax.fori_loop(0, 40, _zrow, 0)

    def _zmat(i, carry):
        for k in range(C // 16):
            zmat_v[i, pl.ds(k * 16, 16)] = jnp.zeros((16,), jnp.float32)
        return carry

    lax.fori_loop(0, 104, _zmat, 0)

    for k in range(6):
        pltpu.async_copy(zmat_v, agg_sh.at[pl.ds(sid * S_CHUNK + k * 104, 104)], sem_g0)
    pltpu.async_copy(zrow_v.at[pl.ds(0, S_CHUNK)],
                     s_sh.at[pl.ds(sid * S_CHUNK, S_CHUNK)], sem_g0)

    @pl.when(sid == NUM_SUBCORES - 1)
    def _():
        pltpu.async_copy(zmat_v.at[pl.ds(0, S_TAIL)],
                         agg_sh.at[pl.ds(NUM_SUBCORES * S_CHUNK, S_TAIL)], sem_g0)
        pltpu.async_copy(zrow_v.at[pl.ds(0, S_TAIL)],
                         s_sh.at[pl.ds(NUM_SUBCORES * S_CHUNK, S_TAIL)], sem_g0)

    for k in range(6):
        pltpu.make_async_copy(
            zmat_v, agg_sh.at[pl.ds(sid * S_CHUNK + k * 104, 104)], sem_g0).wait()
    pltpu.make_async_copy(zrow_v.at[pl.ds(0, S_CHUNK)],
                          s_sh.at[pl.ds(sid * S_CHUNK, S_CHUNK)], sem_g0).wait()

    @pl.when(sid == NUM_SUBCORES - 1)
    def _():
        pltpu.make_async_copy(
            zmat_v.at[pl.ds(0, S_TAIL)],
            agg_sh.at[pl.ds(NUM_SUBCORES * S_CHUNK, S_TAIL)], sem_g0).wait()
        pltpu.make_async_copy(
            zrow_v.at[pl.ds(0, S_TAIL)],
            s_sh.at[pl.ds(NUM_SUBCORES * S_CHUNK, S_TAIL)], sem_g0).wait()

    pltpu.sync_copy(bvec_h, bvec_v)
    plsc.subcore_barrier()

    bvec = bvec_v[...]
    sems_g = (sem_g0, sem_g1)
    sems_s = (sem_s0, sem_s1)
    sems_i = (sem_i0, sem_i1)

    def _fire_idx(cid, nb):
        pltpu.async_copy(epk_h.at[cid, pl.ds(0, 3)], e3_v.at[nb], sems_i[nb])

    def _wait_idx(cid, nb):
        pltpu.make_async_copy(epk_h.at[cid, pl.ds(0, 3)], e3_v.at[nb], sems_i[nb]).wait()

    def _prep_and_fire_gathers(nb):
        for g in range(CH // 16):
            ds = pl.ds(g * 16, 16)
            ty16 = e3_v[nb, 2, ds]
            h16 = e3_v[nb, 0, ds]
            hsc_v[nb, ds] = h16
            iu_v[nb, ds] = h16 * 32 + ty16
            iw_v[nb, ds] = e3_v[nb, 1, ds] * 32 + (ty16 + R)
        pltpu.async_copy(uvb_h.at[iu_v.at[nb]], u_v.at[nb], sems_g[nb])
        pltpu.async_copy(uvb_h.at[iw_v.at[nb]], w_v.at[nb], sems_g[nb])
        pltpu.async_copy(x_h.at[e3_v.at[nb, 1]], xr_v.at[nb], sems_g[nb])

    _fire_idx(wid, 0)
    _wait_idx(wid, 0)
    _prep_and_fire_gathers(0)

    @pl.when(wid + NW < NUM_CHUNKS)
    def _():
        _fire_idx(wid + NW, 1)

    def _pair(i2, carry):
        for b in (0, 1):
            k = i2 * 2 + b
            nb = 1 - b
            cid = wid + k * NW

            @pl.when((k >= 1) & (cid - NW < NUM_CHUNKS))
            def _():
                pltpu.make_async_copy(
                    xr_v.at[nb], agg_sh.at[hsc_v.at[nb]], sems_s[nb]).wait()
                pltpu.make_async_copy(
                    ex_v.at[nb], s_sh.at[hsc_v.at[nb]], sems_s[nb]).wait()

            @pl.when(cid + NW < NUM_CHUNKS)
            def _():
                _wait_idx(cid + NW, nb)
                _prep_and_fire_gathers(nb)

            @pl.when(cid < NUM_CHUNKS)
            def _():
                pltpu.make_async_copy(uvb_h.at[iu_v.at[b]], u_v.at[b], sems_g[b]).wait()
                pltpu.make_async_copy(uvb_h.at[iw_v.at[b]], w_v.at[b], sems_g[b]).wait()
                pltpu.make_async_copy(x_h.at[e3_v.at[b, 1]], xr_v.at[b], sems_g[b]).wait()

                @pl.when(cid + 2 * NW < NUM_CHUNKS)
                def _():
                    _fire_idx(cid + 2 * NW, b)

                for g in range(CH // 16):
                    ds = pl.ds(g * 16, 16)
                    ein = u_v[b, ds] + w_v[b, ds]
                    e = jnp.where(ein >= 0.0, ein, ein * 0.2)
                    ex_v[b, ds] = jnp.exp(e - bvec)

                def _scale(g, carry2):
                    ex16 = ex_v[b, pl.ds(g * 16, 16)]
                    for j in range(16):
                        av = jnp.broadcast_to(ex16[j], (16,))
                        row = g * 16 + j
                        for kk in range(C // 16):
                            xr_v[b, row, pl.ds(kk * 16, 16)] = (
                                xr_v[b, row, pl.ds(kk * 16, 16)] * av)
                    return carry2

                lax.fori_loop(0, CH // 16, _scale, 0)

                pltpu.async_copy(xr_v.at[b], agg_sh.at[hsc_v.at[b]], sems_s[b], add=True)
                pltpu.async_copy(ex_v.at[b], s_sh.at[hsc_v.at[b]], sems_s[b], add=True)

        return carry

    lax.fori_loop(0, (CHUNKS_PER_W + 2) // 2, _pair, 0)
    plsc.subcore_barrier()

    pltpu.async_copy(agg_sh.at[pl.ds(sid * S_CHUNK, S_CHUNK)],
                     agg_o.at[cid_ax, pl.ds(sid * S_CHUNK, S_CHUNK)], sem_g0)
    pltpu.sync_copy(s_sh.at[pl.ds(sid * S_CHUNK, S_CHUNK)], zrow_v.at[pl.ds(0, S_CHUNK)])
    pltpu.sync_copy(zrow_v.at[pl.ds(0, S_CHUNK)],
                    s_o.at[pl.ds(cid_ax * N + sid * S_CHUNK, S_CHUNK)])
    pltpu.make_async_copy(agg_sh.at[pl.ds(sid * S_CHUNK, S_CHUNK)],
                          agg_o.at[cid_ax, pl.ds(sid * S_CHUNK, S_CHUNK)], sem_g0).wait()

    @pl.when(sid == NUM_SUBCORES - 1)
    def _():
        pltpu.sync_copy(agg_sh.at[pl.ds(NUM_SUBCORES * S_CHUNK, S_TAIL)],
                        agg_o.at[cid_ax, pl.ds(NUM_SUBCORES * S_CHUNK, S_TAIL)])
        pltpu.sync_copy(s_sh.at[pl.ds(NUM_SUBCORES * S_CHUNK, S_TAIL)],
                        zrow_v.at[pl.ds(0, S_TAIL)])
        pltpu.sync_copy(zrow_v.at[pl.ds(0, S_TAIL)],
                        s_o.at[pl.ds(cid_ax * N + NUM_SUBCORES * S_CHUNK, S_TAIL)])


_sc_edge = functools.partial(
    pl.kernel,
    mesh=plsc.VectorSubcoreMesh(core_axis_name="c", subcore_axis_name="s"),
    out_type=[
        jax.ShapeDtypeStruct((NUM_CORES, N, C), jnp.float32),
        jax.ShapeDtypeStruct((NUM_CORES * N,), jnp.float32),
    ],
    scratch_types=[
        pltpu.VMEM((640,), jnp.float32),
        pltpu.VMEM((104, C), jnp.float32),
        pltpu.VMEM((2, 3, CH), jnp.int32),
        pltpu.VMEM((2, CH), jnp.int32),
        pltpu.VMEM((2, CH), jnp.int32),
        pltpu.VMEM((2, CH), jnp.int32),
        pltpu.VMEM((2, CH), jnp.float32),
        pltpu.VMEM((2, CH), jnp.float32),
        pltpu.VMEM((2, CH, C), jnp.float32),
        pltpu.VMEM((2, CH), jnp.float32),
        pltpu.VMEM((R,), jnp.float32),
        pltpu.SemaphoreType.DMA,
        pltpu.SemaphoreType.DMA,
        pltpu.SemaphoreType.DMA,
        pltpu.SemaphoreType.DMA,
        pltpu.SemaphoreType.DMA,
        pltpu.SemaphoreType.DMA,
        pltpu.VMEM_SHARED((N, C), jnp.float32),
        pltpu.VMEM_SHARED((N,), jnp.float32),
    ],
)(_sc_edge_body)


def _prologue_body(x_ref, rel_ref, fw_ref, fb_ref,
                   uvb_ref, bv_ref, w_ref_o, bcp_ref_o):
    rel = rel_ref[...]
    fw = fw_ref[...]
    wh = jnp.dot(rel, fw[:, :C], preferred_element_type=jnp.float32)
    wt = jnp.dot(rel, fw[:, C:], preferred_element_type=jnp.float32)
    w = jnp.concatenate([wh, wt], axis=0).T
    bc = jnp.dot(rel, fb_ref[...].reshape(C, 1),
                 preferred_element_type=jnp.float32).T
    bcp = jnp.concatenate([bc, jnp.zeros((1, R), jnp.float32)], axis=1)
    uvb = jnp.dot(x_ref[...], w, preferred_element_type=jnp.float32) + bcp
    uvb_ref[...] = uvb
    w_ref_o[...] = w
    bcp_ref_o[...] = bcp
    b = jnp.max(uvb[:, :R]) + jnp.max(uvb[:, R:])
    bv_ref[...] = jnp.full((1, R), 0.0, jnp.float32) + b


_prologue = pl.pallas_call(
    _prologue_body,
    out_shape=[
        jax.ShapeDtypeStruct((N, 2 * R), jnp.float32),
        jax.ShapeDtypeStruct((1, R), jnp.float32),
        jax.ShapeDtypeStruct((C, 2 * R), jnp.float32),
        jax.ShapeDtypeStruct((1, 2 * R), jnp.float32),
    ],
)


def _update_body(with_uv, agg_ref, s_ref, x_ref, w_ref, bcp_ref, xo_ref, *rest):
    a = agg_ref[...]
    s = s_ref[...]
    ssum = s[0] + s[1]
    agg = a[0] + a[1]
    denom = jnp.where(ssum > 0.0, ssum, 1.0)
    row = jnp.where(ssum > 0.0, agg / denom, 0.0) + x_ref[...]
    nrm = jnp.sqrt(jnp.sum(row * row, axis=1, keepdims=True))
    xo = row / jnp.maximum(nrm, 1e-12)
    xo_ref[...] = xo
    if with_uv:
        uvb_ref, bv_ref = rest
        uvb = jnp.dot(xo, w_ref[...], preferred_element_type=jnp.float32) + bcp_ref[...]
        uvb_ref[...] = uvb
        b = jnp.max(uvb[:, :R]) + jnp.max(uvb[:, R:])
        bv_ref[...] = jnp.full((1, R), 0.0, jnp.float32) + b


_update_mid = pl.pallas_call(
    functools.partial(_update_body, True),
    out_shape=[
        jax.ShapeDtypeStruct((N, C), jnp.float32),
        jax.ShapeDtypeStruct((N, 2 * R), jnp.float32),
        jax.ShapeDtypeStruct((1, R), jnp.float32),
    ],
)

_update_last = pl.pallas_call(
    functools.partial(_update_body, False),
    out_shape=[jax.ShapeDtypeStruct((N, C), jnp.float32)],
)


def kernel(entity_emb, relation_emb, fc_w, fc_b, edge_index, edge_type):
    epk = jnp.concatenate(
        [edge_index.reshape(2, NUM_CHUNKS, 1, CH).transpose(1, 0, 2, 3)
         .reshape(NUM_CHUNKS, 2, CH),
         edge_type.reshape(NUM_CHUNKS, 1, CH),
         jnp.zeros((NUM_CHUNKS, 5, CH), jnp.int32)], axis=1)

    uvb, bv, w, bcp = _prologue(entity_emb, relation_emb, fc_w, fc_b)

    agg2, s2 = _sc_edge(epk, entity_emb, uvb.reshape(N * 2 * R), bv.reshape(R))
    x1, uvb, bv = _update_mid(agg2, s2.reshape(NUM_CORES, N, 1), entity_emb, w, bcp)

    agg2, s2 = _sc_edge(epk, x1, uvb.reshape(N * 2 * R), bv.reshape(R))
    (x2,) = _update_last(agg2, s2.reshape(NUM_CORES, N, 1), x1, w, bcp)
    return x2


# speedup vs baseline: 36.3087x; 1.0675x over previous
import functools

import jax
import jax.numpy as jnp
from jax import lax
from jax.experimental import pallas as pl
from jax.experimental.pallas import tpu as pltpu
from jax.experimental.pallas import tpu_sc as plsc

N = 10000
E = 320000
C = 128
R = 16

NUM_CORES = 2
NUM_SUBCORES = 16
NW = NUM_CORES * NUM_SUBCORES
CH = 128
NUM_CHUNKS = E // CH
CHUNKS_PER_W = -(-NUM_CHUNKS // NW)
S_CHUNK = 624
S_TAIL = N - NUM_SUBCORES * S_CHUNK


def _sc_edge_body(epk_h, x_h, uvb_h, bvec_h,
                  agg_o, s_o,
                  zrow_v, zmat_v, e3_v, hsc_v, iu_v, iw_v, u_v, w_v,
                  xr_v, ex_v, bvec_v, sem_g0, sem_g1, sem_s0, sem_s1,
                  sem_i0, sem_i1, agg_sh, s_sh):
    cid_ax = lax.axis_index("c")
    sid = lax.axis_index("s")
    wid = sid * NUM_CORES + cid_ax

    def _zrow(i, carry):
        zrow_v[pl.ds(i * 16, 16)] = jnp.zeros((16,), jnp.float32)
        return carry

    lax.fori_loop(0, 40, _zrow, 0)

    def _zmat(i, carry):
        for k in range(C // 16):
            zmat_v[i, pl.ds(k * 16, 16)] = jnp.zeros((16,), jnp.float32)
        return carry

    lax.fori_loop(0, 104, _zmat, 0)

    for k in range(6):
        pltpu.async_copy(zmat_v, agg_sh.at[pl.ds(sid * S_CHUNK + k * 104, 104)], sem_g0)
    pltpu.async_copy(zrow_v.at[pl.ds(0, S_CHUNK)],
                     s_sh.at[pl.ds(sid * S_CHUNK, S_CHUNK)], sem_g0)

    @pl.when(sid == NUM_SUBCORES - 1)
    def _():
        pltpu.async_copy(zmat_v.at[pl.ds(0, S_TAIL)],
                         agg_sh.at[pl.ds(NUM_SUBCORES * S_CHUNK, S_TAIL)], sem_g0)
        pltpu.async_copy(zrow_v.at[pl.ds(0, S_TAIL)],
                         s_sh.at[pl.ds(NUM_SUBCORES * S_CHUNK, S_TAIL)], sem_g0)

    for k in range(6):
        pltpu.make_async_copy(
            zmat_v, agg_sh.at[pl.ds(sid * S_CHUNK + k * 104, 104)], sem_g0).wait()
    pltpu.make_async_copy(zrow_v.at[pl.ds(0, S_CHUNK)],
                          s_sh.at[pl.ds(sid * S_CHUNK, S_CHUNK)], sem_g0).wait()

    @pl.when(sid == NUM_SUBCORES - 1)
    def _():
        pltpu.make_async_copy(
            zmat_v.at[pl.ds(0, S_TAIL)],
            agg_sh.at[pl.ds(NUM_SUBCORES * S_CHUNK, S_TAIL)], sem_g0).wait()
        pltpu.make_async_copy(
            zrow_v.at[pl.ds(0, S_TAIL)],
            s_sh.at[pl.ds(NUM_SUBCORES * S_CHUNK, S_TAIL)], sem_g0).wait()

    pltpu.sync_copy(bvec_h, bvec_v)
    plsc.subcore_barrier()

    bvec = bvec_v[...]
    sems_g = (sem_g0, sem_g1)
    sems_s = (sem_s0, sem_s1)
    sems_i = (sem_i0, sem_i1)

    def _fire_idx(cid, nb):
        pltpu.async_copy(epk_h.at[cid, pl.ds(0, 3)], e3_v.at[nb], sems_i[nb])

    def _wait_idx(cid, nb):
        pltpu.make_async_copy(epk_h.at[cid, pl.ds(0, 3)], e3_v.at[nb], sems_i[nb]).wait()

    def _prep_and_fire_gathers(nb):
        for g in range(CH // 16):
            ds = pl.ds(g * 16, 16)
            ty16 = e3_v[nb, 2, ds]
            h16 = e3_v[nb, 0, ds]
            hsc_v[nb, ds] = h16
            iu_v[nb, ds] = h16 * 32 + ty16
            iw_v[nb, ds] = e3_v[nb, 1, ds] * 32 + (ty16 + R)
        pltpu.async_copy(uvb_h.at[iu_v.at[nb]], u_v.at[nb], sems_g[nb])
        pltpu.async_copy(uvb_h.at[iw_v.at[nb]], w_v.at[nb], sems_g[nb])
        pltpu.async_copy(x_h.at[e3_v.at[nb, 1]], xr_v.at[nb], sems_g[nb])

    _fire_idx(wid, 0)
    _wait_idx(wid, 0)
    _prep_and_fire_gathers(0)

    @pl.when(wid + NW < NUM_CHUNKS)
    def _():
        _fire_idx(wid + NW, 1)

    def _pair(i2, carry):
        for b in (0, 1):
            k = i2 * 2 + b
            nb = 1 - b
            cid = wid + k * NW

            @pl.when((k >= 1) & (cid - NW < NUM_CHUNKS))
            def _():
                pltpu.make_async_copy(
                    xr_v.at[nb], agg_sh.at[hsc_v.at[nb]], sems_s[nb]).wait()
                pltpu.make_async_copy(
                    ex_v.at[nb], s_sh.at[hsc_v.at[nb]], sems_s[nb]).wait()

            @pl.when(cid + NW < NUM_CHUNKS)
            def _():
                _wait_idx(cid + NW, nb)
                _prep_and_fire_gathers(nb)

            @pl.when(cid < NUM_CHUNKS)
            def _():
                pltpu.make_async_copy(uvb_h.at[iu_v.at[b]], u_v.at[b], sems_g[b]).wait()
                pltpu.make_async_copy(uvb_h.at[iw_v.at[b]], w_v.at[b], sems_g[b]).wait()
                pltpu.make_async_copy(x_h.at[e3_v.at[b, 1]], xr_v.at[b], sems_g[b]).wait()

                @pl.when(cid + 2 * NW < NUM_CHUNKS)
                def _():
                    _fire_idx(cid + 2 * NW, b)

                for g in range(CH // 16):
                    ds = pl.ds(g * 16, 16)
                    ein = u_v[b, ds] + w_v[b, ds]
                    e = jnp.where(ein >= 0.0, ein, ein * 0.2)
                    ex_v[b, ds] = jnp.exp(e - bvec)

                def _scale(g, carry2):
                    ex16 = ex_v[b, pl.ds(g * 16, 16)]
                    for j in range(16):
                        av = jnp.broadcast_to(ex16[j], (16,))
                        row = g * 16 + j
                        for kk in range(C // 16):
                            xr_v[b, row, pl.ds(kk * 16, 16)] = (
                                xr_v[b, row, pl.ds(kk * 16, 16)] * av)
                    return carry2

                lax.fori_loop(0, CH // 16, _scale, 0)

                pltpu.async_copy(xr_v.at[b], agg_sh.at[hsc_v.at[b]], sems_s[b], add=True)
                pltpu.async_copy(ex_v.at[b], s_sh.at[hsc_v.at[b]], sems_s[b], add=True)

        return carry

    lax.fori_loop(0, (CHUNKS_PER_W + 2) // 2, _pair, 0)
    plsc.subcore_barrier()

    pltpu.async_copy(agg_sh.at[pl.ds(sid * S_CHUNK, S_CHUNK)],
                     agg_o.at[cid_ax, pl.ds(sid * S_CHUNK, S_CHUNK)], sem_g0)
    pltpu.sync_copy(s_sh.at[pl.ds(sid * S_CHUNK, S_CHUNK)], zrow_v.at[pl.ds(0, S_CHUNK)])
    pltpu.sync_copy(zrow_v.at[pl.ds(0, S_CHUNK)],
                    s_o.at[pl.ds(cid_ax * N + sid * S_CHUNK, S_CHUNK)])
    pltpu.make_async_copy(agg_sh.at[pl.ds(sid * S_CHUNK, S_CHUNK)],
                          agg_o.at[cid_ax, pl.ds(sid * S_CHUNK, S_CHUNK)], sem_g0).wait()

    @pl.when(sid == NUM_SUBCORES - 1)
    def _():
        pltpu.sync_copy(agg_sh.at[pl.ds(NUM_SUBCORES * S_CHUNK, S_TAIL)],
                        agg_o.at[cid_ax, pl.ds(NUM_SUBCORES * S_CHUNK, S_TAIL)])
        pltpu.sync_copy(s_sh.at[pl.ds(NUM_SUBCORES * S_CHUNK, S_TAIL)],
                        zrow_v.at[pl.ds(0, S_TAIL)])
        pltpu.sync_copy(zrow_v.at[pl.ds(0, S_TAIL)],
                        s_o.at[pl.ds(cid_ax * N + NUM_SUBCORES * S_CHUNK, S_TAIL)])


_sc_edge = functools.partial(
    pl.kernel,
    mesh=plsc.VectorSubcoreMesh(core_axis_name="c", subcore_axis_name="s"),
    out_type=[
        jax.ShapeDtypeStruct((NUM_CORES, N, C), jnp.float32),
        jax.ShapeDtypeStruct((NUM_CORES * N,), jnp.float32),
    ],
    scratch_types=[
        pltpu.VMEM((640,), jnp.float32),
        pltpu.VMEM((104, C), jnp.float32),
        pltpu.VMEM((2, 3, CH), jnp.int32),
        pltpu.VMEM((2, CH), jnp.int32),
        pltpu.VMEM((2, CH), jnp.int32),
        pltpu.VMEM((2, CH), jnp.int32),
        pltpu.VMEM((2, CH), jnp.float32),
        pltpu.VMEM((2, CH), jnp.float32),
        pltpu.VMEM((2, CH, C), jnp.float32),
        pltpu.VMEM((2, CH), jnp.float32),
        pltpu.VMEM((R,), jnp.float32),
        pltpu.SemaphoreType.DMA,
        pltpu.SemaphoreType.DMA,
        pltpu.SemaphoreType.DMA,
        pltpu.SemaphoreType.DMA,
        pltpu.SemaphoreType.DMA,
        pltpu.SemaphoreType.DMA,
        pltpu.VMEM_SHARED((N, C), jnp.float32),
        pltpu.VMEM_SHARED((N,), jnp.float32),
    ],
)(_sc_edge_body)


def _prologue_body(x_ref, rel_ref, fw_ref, fb_ref,
                   uvb_ref, bv_ref, w_ref_o, bcp_ref_o):
    rel = rel_ref[...]
    fw = fw_ref[...]
    wh = jnp.dot(rel, fw[:, :C], preferred_element_type=jnp.float32)
    wt = jnp.dot(rel, fw[:, C:], preferred_element_type=jnp.float32)
    w = jnp.concatenate([wh, wt], axis=0).T
    bc = jnp.dot(rel, fb_ref[...].reshape(C, 1),
                 preferred_element_type=jnp.float32).T
    bcp = jnp.concatenate([bc, jnp.zeros((1, R), jnp.float32)], axis=1)
    uvb = jnp.dot(x_ref[...], w, preferred_element_type=jnp.float32) + bcp
    uvb_ref[...] = uvb
    w_ref_o[...] = w
    bcp_ref_o[...] = bcp
    b = jnp.max(uvb[:, :R]) + jnp.max(uvb[:, R:])
    bv_ref[...] = jnp.full((1, R), 0.0, jnp.float32) + b


_prologue = pl.pallas_call(
    _prologue_body,
    out_shape=[
        jax.ShapeDtypeStruct((N, 2 * R), jnp.float32),
        jax.ShapeDtypeStruct((1, R), jnp.float32),
        jax.ShapeDtypeStruct((C, 2 * R), jnp.float32),
        jax.ShapeDtypeStruct((1, 2 * R), jnp.float32),
    ],
)


def _update_body(with_uv, agg_ref, s_ref, x_ref, w_ref, bcp_ref, xo_ref, *rest):
    a = agg_ref[...]
    s = s_ref[...]
    ssum = jnp.transpose(s[0:1] + s[1:2])
    agg = a[0] + a[1]
    denom = jnp.where(ssum > 0.0, ssum, 1.0)
    row = jnp.where(ssum > 0.0, agg / denom, 0.0) + x_ref[...]
    nrm = jnp.sqrt(jnp.sum(row * row, axis=1, keepdims=True))
    xo = row / jnp.maximum(nrm, 1e-12)
    xo_ref[...] = xo
    if with_uv:
        uvb_ref, bv_ref = rest
        uvb = jnp.dot(xo, w_ref[...], preferred_element_type=jnp.float32) + bcp_ref[...]
        uvb_ref[...] = uvb
        b = jnp.max(uvb[:, :R]) + jnp.max(uvb[:, R:])
        bv_ref[...] = jnp.full((1, R), 0.0, jnp.float32) + b


_update_mid = pl.pallas_call(
    functools.partial(_update_body, True),
    out_shape=[
        jax.ShapeDtypeStruct((N, C), jnp.float32),
        jax.ShapeDtypeStruct((N, 2 * R), jnp.float32),
        jax.ShapeDtypeStruct((1, R), jnp.float32),
    ],
)

_update_last = pl.pallas_call(
    functools.partial(_update_body, False),
    out_shape=[jax.ShapeDtypeStruct((N, C), jnp.float32)],
)


def kernel(entity_emb, relation_emb, fc_w, fc_b, edge_index, edge_type):
    epk = jnp.concatenate(
        [edge_index.reshape(2, NUM_CHUNKS, 1, CH).transpose(1, 0, 2, 3)
         .reshape(NUM_CHUNKS, 2, CH),
         edge_type.reshape(NUM_CHUNKS, 1, CH)], axis=1)

    uvb, bv, w, bcp = _prologue(entity_emb, relation_emb, fc_w, fc_b)

    agg2, s2 = _sc_edge(epk, entity_emb, uvb.reshape(N * 2 * R), bv.reshape(R))
    x1, uvb, bv = _update_mid(agg2, s2.reshape(NUM_CORES, N), entity_emb, w, bcp)

    agg2, s2 = _sc_edge(epk, x1, uvb.reshape(N * 2 * R), bv.reshape(R))
    (x2,) = _update_last(agg2, s2.reshape(NUM_CORES, N), x1, w, bcp)
    return x2


# speedup vs baseline: 36.3465x; 1.0010x over previous
import functools

import jax
import jax.numpy as jnp
from jax import lax
from jax.experimental import pallas as pl
from jax.experimental.pallas import tpu as pltpu
from jax.experimental.pallas import tpu_sc as plsc

N = 10000
E = 320000
C = 128
R = 16

NUM_CORES = 2
NUM_SUBCORES = 16
NW = NUM_CORES * NUM_SUBCORES
CH = 128
NUM_CHUNKS = E // CH
CHUNKS_PER_W = -(-NUM_CHUNKS // NW)
S_CHUNK = 624
S_TAIL = N - NUM_SUBCORES * S_CHUNK


def _sc_edge_body(epk_h, x_h, uvb_h, bvec_h,
                  agg_o, s_o,
                  zrow_v, zmat_v, e3_v, hsc_v, iu_v, iw_v, u_v, w_v,
                  xr_v, ex_v, bvec_v, sem_g0, sem_g1, sem_s0, sem_s1,
                  sem_i0, sem_i1, agg_sh, s_sh):
    cid_ax = lax.axis_index("c")
    sid = lax.axis_index("s")
    wid = sid * NUM_CORES + cid_ax

    def _zrow(i, carry):
        zrow_v[pl.ds(i * 16, 16)] = jnp.zeros((16,), jnp.float32)
        return carry

    lax.fori_loop(0, 40, _zrow, 0)

    def _zmat(i, carry):
        for k in range(C // 16):
            zmat_v[i, pl.ds(k * 16, 16)] = jnp.zeros((16,), jnp.float32)
        return carry

    lax.fori_loop(0, 104, _zmat, 0)

    for k in range(6):
        pltpu.async_copy(zmat_v, agg_sh.at[pl.ds(sid * S_CHUNK + k * 104, 104)], sem_g0)
    pltpu.async_copy(zrow_v.at[pl.ds(0, S_CHUNK)],
                     s_sh.at[pl.ds(sid * S_CHUNK, S_CHUNK)], sem_g0)

    @pl.when(sid == NUM_SUBCORES - 1)
    def _():
        pltpu.async_copy(zmat_v.at[pl.ds(0, S_TAIL)],
                         agg_sh.at[pl.ds(NUM_SUBCORES * S_CHUNK, S_TAIL)], sem_g0)
        pltpu.async_copy(zrow_v.at[pl.ds(0, S_TAIL)],
                         s_sh.at[pl.ds(NUM_SUBCORES * S_CHUNK, S_TAIL)], sem_g0)

    for k in range(6):
        pltpu.make_async_copy(
            zmat_v, agg_sh.at[pl.ds(sid * S_CHUNK + k * 104, 104)], sem_g0).wait()
    pltpu.make_async_copy(zrow_v.at[pl.ds(0, S_CHUNK)],
                          s_sh.at[pl.ds(sid * S_CHUNK, S_CHUNK)], sem_g0).wait()

    @pl.when(sid == NUM_SUBCORES - 1)
    def _():
        pltpu.make_async_copy(
            zmat_v.at[pl.ds(0, S_TAIL)],
            agg_sh.at[pl.ds(NUM_SUBCORES * S_CHUNK, S_TAIL)], sem_g0).wait()
        pltpu.make_async_copy(
            zrow_v.at[pl.ds(0, S_TAIL)],
            s_sh.at[pl.ds(NUM_SUBCORES * S_CHUNK, S_TAIL)], sem_g0).wait()

    pltpu.sync_copy(bvec_h, bvec_v)
    plsc.subcore_barrier()

    bvec = bvec_v[...]
    sems_g = (sem_g0, sem_g1)
    sems_s = (sem_s0, sem_s1)
    sems_i = (sem_i0, sem_i1)

    def _fire_idx(cid, nb):
        pltpu.async_copy(epk_h.at[cid, pl.ds(0, 3)], e3_v.at[nb], sems_i[nb])

    def _wait_idx(cid, nb):
        pltpu.make_async_copy(epk_h.at[cid, pl.ds(0, 3)], e3_v.at[nb], sems_i[nb]).wait()

    def _prep_and_fire_gathers(nb):
        for g in range(CH // 16):
            ds = pl.ds(g * 16, 16)
            ty16 = e3_v[nb, 2, ds]
            h16 = e3_v[nb, 0, ds]
            hsc_v[nb, ds] = h16
            iu_v[nb, ds] = h16 * 32 + ty16
            iw_v[nb, ds] = e3_v[nb, 1, ds] * 32 + (ty16 + R)
        pltpu.async_copy(uvb_h.at[iu_v.at[nb]], u_v.at[nb], sems_g[nb])
        pltpu.async_copy(uvb_h.at[iw_v.at[nb]], w_v.at[nb], sems_g[nb])
        pltpu.async_copy(x_h.at[e3_v.at[nb, 1]], xr_v.at[nb], sems_g[nb])

    _fire_idx(wid, 0)
    _wait_idx(wid, 0)
    _prep_and_fire_gathers(0)

    @pl.when(wid + NW < NUM_CHUNKS)
    def _():
        _fire_idx(wid + NW, 1)

    def _pair(i2, carry):
        for b in (0, 1):
            k = i2 * 2 + b
            nb = 1 - b
            cid = wid + k * NW

            @pl.when((k >= 1) & (cid - NW < NUM_CHUNKS))
            def _():
                pltpu.make_async_copy(
                    xr_v.at[nb], agg_sh.at[hsc_v.at[nb]], sems_s[nb]).wait()
                pltpu.make_async_copy(
                    ex_v.at[nb], s_sh.at[hsc_v.at[nb]], sems_s[nb]).wait()

            @pl.when(cid + NW < NUM_CHUNKS)
            def _():
                _wait_idx(cid + NW, nb)
                _prep_and_fire_gathers(nb)

            @pl.when(cid < NUM_CHUNKS)
            def _():
                pltpu.make_async_copy(uvb_h.at[iu_v.at[b]], u_v.at[b], sems_g[b]).wait()
                pltpu.make_async_copy(uvb_h.at[iw_v.at[b]], w_v.at[b], sems_g[b]).wait()
                pltpu.make_async_copy(x_h.at[e3_v.at[b, 1]], xr_v.at[b], sems_g[b]).wait()

                @pl.when(cid + 2 * NW < NUM_CHUNKS)
                def _():
                    _fire_idx(cid + 2 * NW, b)

                for g in range(CH // 16):
                    ds = pl.ds(g * 16, 16)
                    ein = u_v[b, ds] + w_v[b, ds]
                    e = jnp.where(ein >= 0.0, ein, ein * 0.2)
                    ex_v[b, ds] = jnp.exp(e - bvec)

                @plsc.parallel_loop(0, CH // 16, unroll=2)
                def _scale(g):
                    ex16 = ex_v[b, pl.ds(g * 16, 16)]
                    for j in range(16):
                        av = jnp.broadcast_to(ex16[j], (16,))
                        row = g * 16 + j
                        for kk in range(C // 16):
                            xr_v[b, row, pl.ds(kk * 16, 16)] = (
                                xr_v[b, row, pl.ds(kk * 16, 16)] * av)

                pltpu.async_copy(xr_v.at[b], agg_sh.at[hsc_v.at[b]], sems_s[b], add=True)
                pltpu.async_copy(ex_v.at[b], s_sh.at[hsc_v.at[b]], sems_s[b], add=True)

        return carry

    lax.fori_loop(0, (CHUNKS_PER_W + 2) // 2, _pair, 0)
    plsc.subcore_barrier()

    pltpu.async_copy(agg_sh.at[pl.ds(sid * S_CHUNK, S_CHUNK)],
                     agg_o.at[cid_ax, pl.ds(sid * S_CHUNK, S_CHUNK)], sem_g0)
    pltpu.sync_copy(s_sh.at[pl.ds(sid * S_CHUNK, S_CHUNK)], zrow_v.at[pl.ds(0, S_CHUNK)])
    pltpu.sync_copy(zrow_v.at[pl.ds(0, S_CHUNK)],
                    s_o.at[pl.ds(cid_ax * N + sid * S_CHUNK, S_CHUNK)])
    pltpu.make_async_copy(agg_sh.at[pl.ds(sid * S_CHUNK, S_CHUNK)],
                          agg_o.at[cid_ax, pl.ds(sid * S_CHUNK, S_CHUNK)], sem_g0).wait()

    @pl.when(sid == NUM_SUBCORES - 1)
    def _():
        pltpu.sync_copy(agg_sh.at[pl.ds(NUM_SUBCORES * S_CHUNK, S_TAIL)],
                        agg_o.at[cid_ax, pl.ds(NUM_SUBCORES * S_CHUNK, S_TAIL)])
        pltpu.sync_copy(s_sh.at[pl.ds(NUM_SUBCORES * S_CHUNK, S_TAIL)],
                        zrow_v.at[pl.ds(0, S_TAIL)])
        pltpu.sync_copy(zrow_v.at[pl.ds(0, S_TAIL)],
                        s_o.at[pl.ds(cid_ax * N + NUM_SUBCORES * S_CHUNK, S_TAIL)])


_sc_edge = functools.partial(
    pl.kernel,
    mesh=plsc.VectorSubcoreMesh(core_axis_name="c", subcore_axis_name="s"),
    out_type=[
        jax.ShapeDtypeStruct((NUM_CORES, N, C), jnp.float32),
        jax.ShapeDtypeStruct((NUM_CORES * N,), jnp.float32),
    ],
    scratch_types=[
        pltpu.VMEM((640,), jnp.float32),
        pltpu.VMEM((104, C), jnp.float32),
        pltpu.VMEM((2, 3, CH), jnp.int32),
        pltpu.VMEM((2, CH), jnp.int32),
        pltpu.VMEM((2, CH), jnp.int32),
        pltpu.VMEM((2, CH), jnp.int32),
        pltpu.VMEM((2, CH), jnp.float32),
        pltpu.VMEM((2, CH), jnp.float32),
        pltpu.VMEM((2, CH, C), jnp.float32),
        pltpu.VMEM((2, CH), jnp.float32),
        pltpu.VMEM((R,), jnp.float32),
        pltpu.SemaphoreType.DMA,
        pltpu.SemaphoreType.DMA,
        pltpu.SemaphoreType.DMA,
        pltpu.SemaphoreType.DMA,
        pltpu.SemaphoreType.DMA,
        pltpu.SemaphoreType.DMA,
        pltpu.VMEM_SHARED((N, C), jnp.float32),
        pltpu.VMEM_SHARED((N,), jnp.float32),
    ],
)(_sc_edge_body)


def _prologue_body(x_ref, rel_ref, fw_ref, fb_ref,
                   uvb_ref, bv_ref, w_ref_o, bcp_ref_o):
    rel = rel_ref[...]
    fw = fw_ref[...]
    wh = jnp.dot(rel, fw[:, :C], preferred_element_type=jnp.float32)
    wt = jnp.dot(rel, fw[:, C:], preferred_element_type=jnp.float32)
    w = jnp.concatenate([wh, wt], axis=0).T
    bc = jnp.dot(rel, fb_ref[...].reshape(C, 1),
                 preferred_element_type=jnp.float32).T
    bcp = jnp.concatenate([bc, jnp.zeros((1, R), jnp.float32)], axis=1)
    uvb = jnp.dot(x_ref[...], w, preferred_element_type=jnp.float32) + bcp
    uvb_ref[...] = uvb
    w_ref_o[...] = w
    bcp_ref_o[...] = bcp
    b = jnp.max(uvb[:, :R]) + jnp.max(uvb[:, R:])
    bv_ref[...] = jnp.full((1, R), 0.0, jnp.float32) + b


_prologue = pl.pallas_call(
    _prologue_body,
    out_shape=[
        jax.ShapeDtypeStruct((N, 2 * R), jnp.float32),
        jax.ShapeDtypeStruct((1, R), jnp.float32),
        jax.ShapeDtypeStruct((C, 2 * R), jnp.float32),
        jax.ShapeDtypeStruct((1, 2 * R), jnp.float32),
    ],
)


def _update_body(with_uv, agg_ref, s_ref, x_ref, w_ref, bcp_ref, xo_ref, *rest):
    a = agg_ref[...]
    s = s_ref[...]
    ssum = jnp.transpose(s[0:1] + s[1:2])
    agg = a[0] + a[1]
    denom = jnp.where(ssum > 0.0, ssum, 1.0)
    row = jnp.where(ssum > 0.0, agg / denom, 0.0) + x_ref[...]
    nrm = jnp.sqrt(jnp.sum(row * row, axis=1, keepdims=True))
    xo = row / jnp.maximum(nrm, 1e-12)
    xo_ref[...] = xo
    if with_uv:
        uvb_ref, bv_ref = rest
        uvb = jnp.dot(xo, w_ref[...], preferred_element_type=jnp.float32) + bcp_ref[...]
        uvb_ref[...] = uvb
        b = jnp.max(uvb[:, :R]) + jnp.max(uvb[:, R:])
        bv_ref[...] = jnp.full((1, R), 0.0, jnp.float32) + b


_update_mid = pl.pallas_call(
    functools.partial(_update_body, True),
    out_shape=[
        jax.ShapeDtypeStruct((N, C), jnp.float32),
        jax.ShapeDtypeStruct((N, 2 * R), jnp.float32),
        jax.ShapeDtypeStruct((1, R), jnp.float32),
    ],
)

_update_last = pl.pallas_call(
    functools.partial(_update_body, False),
    out_shape=[jax.ShapeDtypeStruct((N, C), jnp.float32)],
)


def kernel(entity_emb, relation_emb, fc_w, fc_b, edge_index, edge_type):
    epk = jnp.concatenate(
        [edge_index.reshape(2, NUM_CHUNKS, 1, CH).transpose(1, 0, 2, 3)
         .reshape(NUM_CHUNKS, 2, CH),
         edge_type.reshape(NUM_CHUNKS, 1, CH)], axis=1)

    uvb, bv, w, bcp = _prologue(entity_emb, relation_emb, fc_w, fc_b)

    agg2, s2 = _sc_edge(epk, entity_emb, uvb.reshape(N * 2 * R), bv.reshape(R))
    x1, uvb, bv = _update_mid(agg2, s2.reshape(NUM_CORES, N), entity_emb, w, bcp)

    agg2, s2 = _sc_edge(epk, x1, uvb.reshape(N * 2 * R), bv.reshape(R))
    (x2,) = _update_last(agg2, s2.reshape(NUM_CORES, N), x1, w, bcp)
    return x2
